# baseline (speedup 1.0000x reference)
import functools

import numpy as np

import jax
import jax.numpy as jnp
from jax import lax
from jax.experimental import pallas as pl
from jax.experimental.pallas import tpu as pltpu
from jax.experimental.pallas import tpu_sc as plsc

B = 1024
D = 128
NC = 2
NS = 16
NW = NC * NS
G = 128
JUNK = B
ACC_ROWS = 1152
RPT = ACC_ROWS // NS
OPT = B // NS


def _sc_aggregate(x_l, e_l, x_r, e_r):
    E = e_l.shape[0] // 2
    EPW = E // NW
    NV = EPW // 16
    MAXM = EPW + G

    mesh = plsc.VectorSubcoreMesh(
        core_axis_name="c", subcore_axis_name="s",
        num_cores=NC, num_subcores=NS)

    @functools.partial(
        pl.kernel,
        out_type=(
            jax.ShapeDtypeStruct((NC * B, D), jnp.float32),
            jax.ShapeDtypeStruct((NW * B,), jnp.float32),
            jax.ShapeDtypeStruct((NC * B, D), jnp.float32),
            jax.ShapeDtypeStruct((NW * B,), jnp.float32),
        ),
        mesh=mesh,
        compiler_params=pltpu.CompilerParams(
            needs_layout_passes=False, use_tc_tiling_on_sc=False),
        scratch_types=[
            pltpu.VMEM((EPW,), jnp.int32),
            pltpu.VMEM((EPW,), jnp.int32),
            pltpu.VMEM((MAXM,), jnp.int32),
            pltpu.VMEM((MAXM,), jnp.int32),
            pltpu.VMEM((G,), jnp.int32),
            pltpu.VMEM((G,), jnp.int32),
            pltpu.VMEM((G, D // 2), jnp.int32),
            pltpu.VMEM((G,), jnp.int32),
            pltpu.VMEM((G,), jnp.int32),
            pltpu.VMEM((G, D // 2), jnp.int32),
            pltpu.VMEM((G,), jnp.int32),
            pltpu.VMEM((G,), jnp.int32),
            pltpu.VMEM((G, D // 2), jnp.int32),
            pltpu.VMEM((G,), jnp.int32),
            pltpu.VMEM((G,), jnp.int32),
            pltpu.VMEM((G, D // 2), jnp.int32),
            pltpu.VMEM((G, D), jnp.float32),
            pltpu.VMEM((B,), jnp.float32),
            pltpu.VMEM_SHARED((ACC_ROWS, D), jnp.float32),
            pltpu.VMEM_SHARED((ACC_ROWS, D), jnp.float32),
            pltpu.SemaphoreType.DMA,
            pltpu.SemaphoreType.DMA,
            pltpu.SemaphoreType.DMA,
            pltpu.SemaphoreType.DMA,
        ],
    )
    def sc_kernel(xl_hbm, el_hbm, xr_hbm, er_hbm,
                  accl_hbm, degl_hbm, accr_hbm, degr_hbm,
                  dstv, srcv, mdst, msrc, gdst0, gsrc0, rows0,
                  gdst1, gsrc1, rows1, gdst2, gsrc2, rows2,
                  gdst3, gsrc3, rows3, fbuf, degv,
                  acc_l, acc_r, gsem0, gsem1, gsem2, gsem3):
        cid = lax.axis_index("c")
        sid = lax.axis_index("s")
        wid = sid * NC + cid

        ones16 = jnp.ones((16,), jnp.float32)
        zeros16 = jnp.zeros((16,), jnp.float32)
        lane15 = jnp.full((16,), 15, jnp.int32)
        bufs = ((gdst0, gsrc0, rows0, gsem0), (gdst1, gsrc1, rows1, gsem1),
                (gdst2, gsrc2, rows2, gsem2), (gdst3, gsrc3, rows3, gsem3))
        NBUF = len(bufs)

        def zrow(i, _):
            fbuf[i // 8, pl.ds((i % 8) * 16, 16)] = zeros16
            return 0

        lax.fori_loop(0, RPT * (D // 16), zrow, 0)
        r0 = sid * RPT
        pltpu.sync_copy(fbuf.at[pl.ds(0, RPT)], acc_l.at[pl.ds(r0, RPT)])
        pltpu.sync_copy(fbuf.at[pl.ds(0, RPT)], acc_r.at[pl.ds(r0, RPT)])
        plsc.subcore_barrier()

        def process(x_hbm, e_hbm, acc_sh, deg_hbm):
            base = wid * EPW
            pltpu.sync_copy(e_hbm.at[pl.ds(E + base, EPW)], dstv)
            pltpu.sync_copy(e_hbm.at[pl.ds(base, EPW)], srcv)

            def zdeg(i, _):
                degv[pl.ds(i * 16, 16)] = zeros16
                return 0

            lax.fori_loop(0, B // 16, zdeg, 0)

            def compact(i, off):
                d = dstv[pl.ds(i * 16, 16)]
                s = srcv[pl.ds(i * 16, 16)]
                mask = d < B
                scan = plsc.cumsum(mask.astype(jnp.int32))
                pos = off + scan - 1
                plsc.store_scatter(mdst, [pos], d, mask=mask)
                plsc.store_scatter(msrc, [pos], s, mask=mask)
                plsc.addupdate_scatter(degv, [d], ones16, mask=mask)
                last = lax.gather(
                    scan, lane15[:, None],
                    lax.GatherDimensionNumbers(
                        offset_dims=(), collapsed_slice_dims=(0,),
                        start_index_map=(0,)),
                    slice_sizes=(1,),
                    mode=lax.GatherScatterMode.PROMISE_IN_BOUNDS)
                return off + last

            off = lax.fori_loop(0, NV, compact, jnp.zeros((16,), jnp.int32),
                                unroll=2)
            m = off[0]

            def pad(j, _):
                mdst[pl.ds(m + j * 16, 16)] = jnp.full((16,), JUNK, jnp.int32)
                msrc[pl.ds(m + j * 16, 16)] = jnp.zeros((16,), jnp.int32)
                return 0

            lax.fori_loop(0, G // 16, pad, 0)

            ng = (m + G - 1) // G

            def stage(g, gd, gs):
                def cpy(j, _):
                    gd[pl.ds(j * 16, 16)] = mdst[pl.ds(g * G + j * 16, 16)]
                    gs[pl.ds(j * 16, 16)] = msrc[pl.ds(g * G + j * 16, 16)]
                    return 0

                lax.fori_loop(0, G // 16, cpy, 0)

            for b in range(NBUF):
                gd, gs, rw, sem = bufs[b]

                @pl.when(b < ng)
                def _():
                    stage(b, gd, gs)
                    pltpu.async_copy(x_hbm.at[gs], rw, sem)

            def ring(p, _):
                for b in range(NBUF):
                    g = NBUF * p + b
                    gd, gs, rw, sem = bufs[b]

                    @pl.when(g < ng)
                    def _():
                        pltpu.make_async_copy(x_hbm.at[gs], rw, sem).wait()

                        def conv(i, _):
                            for c in range(D // 32):
                                w = rw[i, pl.ds(16 * c, 16)]
                                fbuf[i, pl.ds(32 * c, 16)] = plsc.bitcast(
                                    w << 16, jnp.float32)
                                fbuf[i, pl.ds(32 * c + 16, 16)] = plsc.bitcast(
                                    w & jnp.int32(-65536), jnp.float32)
                            return 0

                        lax.fori_loop(0, G, conv, 0)
                        pltpu.sync_copy(fbuf, acc_sh.at[gd], add=True)

                        @pl.when(g + NBUF < ng)
                        def _():
                            stage(g + NBUF, gd, gs)
                            pltpu.async_copy(x_hbm.at[gs], rw, sem)

                return 0

            lax.fori_loop(0, (ng + NBUF - 1) // NBUF, ring, 0)

            pltpu.sync_copy(degv, deg_hbm.at[pl.ds(wid * B, B)])

        process(xl_hbm, el_hbm, acc_l, degl_hbm)
        process(xr_hbm, er_hbm, acc_r, degr_hbm)
        plsc.subcore_barrier()

        o0 = sid * OPT
        ob = cid * B + o0
        pltpu.sync_copy(acc_l.at[pl.ds(o0, OPT)], accl_hbm.at[pl.ds(ob, OPT)])
        pltpu.sync_copy(acc_r.at[pl.ds(o0, OPT)], accr_hbm.at[pl.ds(ob, OPT)])

    return sc_kernel(x_l, e_l, x_r, e_r)


def _tc_body(xl, xr, accl, accr, degl, degr, ws, wn, lw, lb,
             logits_o, dist_o, embl_o, embr_o):
    ones_w = jnp.ones((NW, 1), jnp.float32)
    dims = (((0,), (0,)), ((), ()))
    dl = lax.dot_general(degl[...], ones_w, dims,
                         preferred_element_type=jnp.float32)
    dr = lax.dot_general(degr[...], ones_w, dims,
                         preferred_element_type=jnp.float32)
    aggl = accl[0:B, :] + accl[B:2 * B, :]
    aggr = accr[0:B, :] + accr[B:2 * B, :]
    meanl = aggl / jnp.maximum(dl, 1.0)
    meanr = aggr / jnp.maximum(dr, 1.0)
    embl = jax.nn.relu(
        jnp.dot(xl[...], ws[...], preferred_element_type=jnp.float32)
        + jnp.dot(meanl, wn[...], preferred_element_type=jnp.float32))
    embr = jax.nn.relu(
        jnp.dot(xr[...], ws[...], preferred_element_type=jnp.float32)
        + jnp.dot(meanr, wn[...], preferred_element_type=jnp.float32))
    dot = jnp.sum(embl * embr, axis=1, keepdims=True)
    nl = jnp.maximum(jnp.sqrt(jnp.sum(embl * embl, axis=1, keepdims=True)), 1e-8)
    nr = jnp.maximum(jnp.sqrt(jnp.sum(embr * embr, axis=1, keepdims=True)), 1e-8)
    dist = dot / (nl * nr)
    logits_o[...] = dist * lw[...] + lb[...]
    dist_o[...] = dist
    embl_o[...] = embl
    embr_o[...] = embr


_PERM = np.zeros(D, dtype=np.int32)
for _c in range(D // 32):
    for _k in range(16):
        _PERM[32 * _c + _k] = 32 * _c + 2 * _k
        _PERM[32 * _c + 16 + _k] = 32 * _c + 2 * _k + 1


def kernel(x_l, edge_index_l, x_r, edge_index_r, W_self, W_neigh, lin_W,
           lin_b, batch_size):
    del batch_size
    x_l = x_l.astype(jnp.float32)
    x_r = x_r.astype(jnp.float32)
    N = x_l.shape[0]
    E = edge_index_l.shape[1]
    el = edge_index_l.astype(jnp.int32).reshape(2 * E)
    er = edge_index_r.astype(jnp.int32).reshape(2 * E)

    xb_l = lax.bitcast_convert_type(
        x_l.astype(jnp.bfloat16).reshape(N, D // 2, 2), jnp.int32)
    xb_r = lax.bitcast_convert_type(
        x_r.astype(jnp.bfloat16).reshape(N, D // 2, 2), jnp.int32)

    accl, degl, accr, degr = _sc_aggregate(xb_l, el, xb_r, er)

    full = lambda s: pl.BlockSpec(s, lambda i: (0,) * len(s))
    logits, dist, embl, embr = pl.pallas_call(
        _tc_body,
        grid=(1,),
        out_shape=(
            jax.ShapeDtypeStruct((B, 2), jnp.float32),
            jax.ShapeDtypeStruct((B, 1), jnp.float32),
            jax.ShapeDtypeStruct((B, D), jnp.float32),
            jax.ShapeDtypeStruct((B, D), jnp.float32),
        ),
        out_specs=(full((B, 2)), full((B, 1)), full((B, D)), full((B, D))),
        in_specs=[
            full((B, D)), full((B, D)),
            full((NC * B, D)), full((NC * B, D)),
            full((NW, B)), full((NW, B)),
            full((D, D)), full((D, D)),
            full((1, 2)), full((1, 2)),
        ],
    )(x_l, x_r, accl, accr, degl.reshape(NW, B), degr.reshape(NW, B),
      W_self, W_neigh[_PERM, :], lin_W, lin_b.reshape(1, 2))

    return (logits, dist.reshape(B), embl, embr)


# speedup vs baseline: 1.0926x; 1.0926x over previous
import functools

import numpy as np

import jax
import jax.numpy as jnp
from jax import lax
from jax.experimental import pallas as pl
from jax.experimental.pallas import tpu as pltpu
from jax.experimental.pallas import tpu_sc as plsc

B = 1024
D = 128
NC = 2
NS = 16
NW = NC * NS
G = 128
JUNK = B
ACC_ROWS = 1152
RPT = ACC_ROWS // NS
OPT = B // NS


def _sc_aggregate(x_l, e_l, x_r, e_r):
    E = e_l.shape[0] // 2
    EPW = E // NW
    NV = EPW // 16
    MAXM = EPW + G

    mesh = plsc.VectorSubcoreMesh(
        core_axis_name="c", subcore_axis_name="s",
        num_cores=NC, num_subcores=NS)

    @functools.partial(
        pl.kernel,
        out_type=(
            jax.ShapeDtypeStruct((NC * B, D), jnp.float32),
            jax.ShapeDtypeStruct((NW * B,), jnp.float32),
            jax.ShapeDtypeStruct((NC * B, D), jnp.float32),
            jax.ShapeDtypeStruct((NW * B,), jnp.float32),
        ),
        mesh=mesh,
        compiler_params=pltpu.CompilerParams(
            needs_layout_passes=False, use_tc_tiling_on_sc=False),
        scratch_types=[
            pltpu.VMEM((EPW,), jnp.int32),
            pltpu.VMEM((EPW,), jnp.int32),
            pltpu.VMEM((MAXM,), jnp.int32),
            pltpu.VMEM((MAXM,), jnp.int32),
            pltpu.VMEM((G,), jnp.int32),
            pltpu.VMEM((G,), jnp.int32),
            pltpu.VMEM((G, D // 2), jnp.int32),
            pltpu.VMEM((G,), jnp.int32),
            pltpu.VMEM((G,), jnp.int32),
            pltpu.VMEM((G, D // 2), jnp.int32),
            pltpu.VMEM((G,), jnp.int32),
            pltpu.VMEM((G,), jnp.int32),
            pltpu.VMEM((G, D // 2), jnp.int32),
            pltpu.VMEM((G,), jnp.int32),
            pltpu.VMEM((G,), jnp.int32),
            pltpu.VMEM((G, D // 2), jnp.int32),
            pltpu.VMEM((G, D), jnp.float32),
            pltpu.VMEM((B,), jnp.float32),
            pltpu.VMEM_SHARED((ACC_ROWS, D), jnp.float32),
            pltpu.VMEM_SHARED((ACC_ROWS, D), jnp.float32),
            pltpu.SemaphoreType.DMA,
            pltpu.SemaphoreType.DMA,
            pltpu.SemaphoreType.DMA,
            pltpu.SemaphoreType.DMA,
        ],
    )
    def sc_kernel(xl_hbm, el_hbm, xr_hbm, er_hbm,
                  accl_hbm, degl_hbm, accr_hbm, degr_hbm,
                  dstv, srcv, mdst, msrc, gdst0, gsrc0, rows0,
                  gdst1, gsrc1, rows1, gdst2, gsrc2, rows2,
                  gdst3, gsrc3, rows3, fbuf, degv,
                  acc_l, acc_r, gsem0, gsem1, gsem2, gsem3):
        cid = lax.axis_index("c")
        sid = lax.axis_index("s")
        wid = sid * NC + cid

        ones16 = jnp.ones((16,), jnp.float32)
        zeros16 = jnp.zeros((16,), jnp.float32)
        lane15 = jnp.full((16,), 15, jnp.int32)
        bufs = ((gdst0, gsrc0, rows0, gsem0), (gdst1, gsrc1, rows1, gsem1),
                (gdst2, gsrc2, rows2, gsem2), (gdst3, gsrc3, rows3, gsem3))
        NBUF = len(bufs)

        def zrow(i, _):
            fbuf[i // 8, pl.ds((i % 8) * 16, 16)] = zeros16
            return 0

        lax.fori_loop(0, RPT * (D // 16), zrow, 0)
        r0 = sid * RPT
        pltpu.sync_copy(fbuf.at[pl.ds(0, RPT)], acc_l.at[pl.ds(r0, RPT)])
        pltpu.sync_copy(fbuf.at[pl.ds(0, RPT)], acc_r.at[pl.ds(r0, RPT)])
        plsc.subcore_barrier()

        def process(x_hbm, e_hbm, acc_sh, deg_hbm):
            base = wid * EPW
            pltpu.sync_copy(e_hbm.at[pl.ds(E + base, EPW)], dstv)
            pltpu.sync_copy(e_hbm.at[pl.ds(base, EPW)], srcv)

            def zdeg(i, _):
                degv[pl.ds(i * 16, 16)] = zeros16
                return 0

            lax.fori_loop(0, B // 16, zdeg, 0)

            def compact(i, off):
                d = dstv[pl.ds(i * 16, 16)]
                s = srcv[pl.ds(i * 16, 16)]
                mask = d < B
                scan = plsc.cumsum(mask.astype(jnp.int32))
                pos = off + scan - 1
                plsc.store_scatter(mdst, [pos], d, mask=mask)
                plsc.store_scatter(msrc, [pos], s, mask=mask)
                plsc.addupdate_scatter(degv, [d], ones16, mask=mask)
                last = lax.gather(
                    scan, lane15[:, None],
                    lax.GatherDimensionNumbers(
                        offset_dims=(), collapsed_slice_dims=(0,),
                        start_index_map=(0,)),
                    slice_sizes=(1,),
                    mode=lax.GatherScatterMode.PROMISE_IN_BOUNDS)
                return off + last

            off = lax.fori_loop(0, NV, compact, jnp.zeros((16,), jnp.int32),
                                unroll=2)
            m = off[0]

            def pad(j, _):
                mdst[pl.ds(m + j * 16, 16)] = jnp.full((16,), JUNK, jnp.int32)
                msrc[pl.ds(m + j * 16, 16)] = jnp.zeros((16,), jnp.int32)
                return 0

            lax.fori_loop(0, G // 16, pad, 0)

            ng = (m + G - 1) // G

            def stage(g, gd, gs):
                def cpy(j, _):
                    gd[pl.ds(j * 16, 16)] = mdst[pl.ds(g * G + j * 16, 16)]
                    gs[pl.ds(j * 16, 16)] = msrc[pl.ds(g * G + j * 16, 16)]
                    return 0

                lax.fori_loop(0, G // 16, cpy, 0)

            for b in range(NBUF):
                gd, gs, rw, sem = bufs[b]

                @pl.when(b < ng)
                def _():
                    stage(b, gd, gs)
                    pltpu.async_copy(x_hbm.at[gs], rw, sem)

            def ring(p, _):
                for b in range(NBUF):
                    g = NBUF * p + b
                    gd, gs, rw, sem = bufs[b]

                    @pl.when(g < ng)
                    def _():
                        pltpu.make_async_copy(x_hbm.at[gs], rw, sem).wait()

                        @plsc.parallel_loop(0, G, unroll=4)
                        def conv(i):
                            for c in range(D // 32):
                                w = rw[i, pl.ds(16 * c, 16)]
                                fbuf[i, pl.ds(32 * c, 16)] = plsc.bitcast(
                                    w << 16, jnp.float32)
                                fbuf[i, pl.ds(32 * c + 16, 16)] = plsc.bitcast(
                                    w & jnp.int32(-65536), jnp.float32)
                        pltpu.sync_copy(fbuf, acc_sh.at[gd], add=True)

                        @pl.when(g + NBUF < ng)
                        def _():
                            stage(g + NBUF, gd, gs)
                            pltpu.async_copy(x_hbm.at[gs], rw, sem)

                return 0

            lax.fori_loop(0, (ng + NBUF - 1) // NBUF, ring, 0)

            pltpu.sync_copy(degv, deg_hbm.at[pl.ds(wid * B, B)])

        process(xl_hbm, el_hbm, acc_l, degl_hbm)
        process(xr_hbm, er_hbm, acc_r, degr_hbm)
        plsc.subcore_barrier()

        o0 = sid * OPT
        ob = cid * B + o0
        pltpu.sync_copy(acc_l.at[pl.ds(o0, OPT)], accl_hbm.at[pl.ds(ob, OPT)])
        pltpu.sync_copy(acc_r.at[pl.ds(o0, OPT)], accr_hbm.at[pl.ds(ob, OPT)])

    return sc_kernel(x_l, e_l, x_r, e_r)


def _tc_body(xl, xr, accl, accr, degl, degr, ws, wn, lw, lb,
             logits_o, dist_o, embl_o, embr_o):
    ones_w = jnp.ones((NW, 1), jnp.float32)
    dims = (((0,), (0,)), ((), ()))
    dl = lax.dot_general(degl[...], ones_w, dims,
                         preferred_element_type=jnp.float32)
    dr = lax.dot_general(degr[...], ones_w, dims,
                         preferred_element_type=jnp.float32)
    aggl = accl[0:B, :] + accl[B:2 * B, :]
    aggr = accr[0:B, :] + accr[B:2 * B, :]
    meanl = aggl / jnp.maximum(dl, 1.0)
    meanr = aggr / jnp.maximum(dr, 1.0)
    embl = jax.nn.relu(
        jnp.dot(xl[...], ws[...], preferred_element_type=jnp.float32)
        + jnp.dot(meanl, wn[...], preferred_element_type=jnp.float32))
    embr = jax.nn.relu(
        jnp.dot(xr[...], ws[...], preferred_element_type=jnp.float32)
        + jnp.dot(meanr, wn[...], preferred_element_type=jnp.float32))
    dot = jnp.sum(embl * embr, axis=1, keepdims=True)
    nl = jnp.maximum(jnp.sqrt(jnp.sum(embl * embl, axis=1, keepdims=True)), 1e-8)
    nr = jnp.maximum(jnp.sqrt(jnp.sum(embr * embr, axis=1, keepdims=True)), 1e-8)
    dist = dot / (nl * nr)
    logits_o[...] = dist * lw[...] + lb[...]
    dist_o[...] = dist
    embl_o[...] = embl
    embr_o[...] = embr


_PERM = np.zeros(D, dtype=np.int32)
for _c in range(D // 32):
    for _k in range(16):
        _PERM[32 * _c + _k] = 32 * _c + 2 * _k
        _PERM[32 * _c + 16 + _k] = 32 * _c + 2 * _k + 1


def kernel(x_l, edge_index_l, x_r, edge_index_r, W_self, W_neigh, lin_W,
           lin_b, batch_size):
    del batch_size
    x_l = x_l.astype(jnp.float32)
    x_r = x_r.astype(jnp.float32)
    N = x_l.shape[0]
    E = edge_index_l.shape[1]
    el = edge_index_l.astype(jnp.int32).reshape(2 * E)
    er = edge_index_r.astype(jnp.int32).reshape(2 * E)

    xb_l = lax.bitcast_convert_type(
        x_l.astype(jnp.bfloat16).reshape(N, D // 2, 2), jnp.int32)
    xb_r = lax.bitcast_convert_type(
        x_r.astype(jnp.bfloat16).reshape(N, D // 2, 2), jnp.int32)

    accl, degl, accr, degr = _sc_aggregate(xb_l, el, xb_r, er)

    full = lambda s: pl.BlockSpec(s, lambda i: (0,) * len(s))
    logits, dist, embl, embr = pl.pallas_call(
        _tc_body,
        grid=(1,),
        out_shape=(
            jax.ShapeDtypeStruct((B, 2), jnp.float32),
            jax.ShapeDtypeStruct((B, 1), jnp.float32),
            jax.ShapeDtypeStruct((B, D), jnp.float32),
            jax.ShapeDtypeStruct((B, D), jnp.float32),
        ),
        out_specs=(full((B, 2)), full((B, 1)), full((B, D)), full((B, D))),
        in_specs=[
            full((B, D)), full((B, D)),
            full((NC * B, D)), full((NC * B, D)),
            full((NW, B)), full((NW, B)),
            full((D, D)), full((D, D)),
            full((1, 2)), full((1, 2)),
        ],
    )(x_l, x_r, accl, accr, degl.reshape(NW, B), degr.reshape(NW, B),
      W_self, W_neigh[_PERM, :], lin_W, lin_b.reshape(1, 2))

    return (logits, dist.reshape(B), embl, embr)


# speedup vs baseline: 1.2271x; 1.1231x over previous
import functools

import numpy as np

import jax
import jax.numpy as jnp
from jax import lax
from jax.experimental import pallas as pl
from jax.experimental.pallas import tpu as pltpu
from jax.experimental.pallas import tpu_sc as plsc

B = 1024
D = 128
NC = 2
NS = 16
NW = NC * NS
G = 128
JUNK = B
ACC_ROWS = 1152
RPT = ACC_ROWS // NS
OPT = B // NS


def _sc_aggregate(x_l, e_l, x_r, e_r):
    E = e_l.shape[0] // 2
    EPW = E // NW
    NV = EPW // 16
    MAXM = EPW + G

    mesh = plsc.VectorSubcoreMesh(
        core_axis_name="c", subcore_axis_name="s",
        num_cores=NC, num_subcores=NS)

    @functools.partial(
        pl.kernel,
        out_type=(
            jax.ShapeDtypeStruct((NC * B, D), jnp.float32),
            jax.ShapeDtypeStruct((NW * B,), jnp.float32),
            jax.ShapeDtypeStruct((NC * B, D), jnp.float32),
            jax.ShapeDtypeStruct((NW * B,), jnp.float32),
        ),
        mesh=mesh,
        compiler_params=pltpu.CompilerParams(needs_layout_passes=False),
        scratch_types=[
            pltpu.VMEM((EPW,), jnp.int32),
            pltpu.VMEM((EPW,), jnp.int32),
            pltpu.VMEM((MAXM,), jnp.int32),
            pltpu.VMEM((MAXM,), jnp.int32),
            pltpu.VMEM((G,), jnp.int32),
            pltpu.VMEM((G,), jnp.int32),
            pltpu.VMEM((G, D), jnp.float32),
            pltpu.VMEM((G,), jnp.int32),
            pltpu.VMEM((G,), jnp.int32),
            pltpu.VMEM((G, D), jnp.float32),
            pltpu.VMEM((G,), jnp.int32),
            pltpu.VMEM((G,), jnp.int32),
            pltpu.VMEM((G, D), jnp.float32),
            pltpu.VMEM((G,), jnp.int32),
            pltpu.VMEM((G,), jnp.int32),
            pltpu.VMEM((G, D), jnp.float32),
            pltpu.VMEM((B,), jnp.float32),
            pltpu.VMEM_SHARED((ACC_ROWS, D), jnp.float32),
            pltpu.VMEM_SHARED((ACC_ROWS, D), jnp.float32),
            pltpu.SemaphoreType.DMA,
            pltpu.SemaphoreType.DMA,
            pltpu.SemaphoreType.DMA,
            pltpu.SemaphoreType.DMA,
        ],
    )
    def sc_kernel(xl_hbm, el_hbm, xr_hbm, er_hbm,
                  accl_hbm, degl_hbm, accr_hbm, degr_hbm,
                  dstv, srcv, mdst, msrc, gdst0, gsrc0, rows0,
                  gdst1, gsrc1, rows1, gdst2, gsrc2, rows2,
                  gdst3, gsrc3, rows3, degv,
                  acc_l, acc_r, gsem0, gsem1, gsem2, gsem3):
        cid = lax.axis_index("c")
        sid = lax.axis_index("s")
        wid = sid * NC + cid

        ones16 = jnp.ones((16,), jnp.float32)
        zeros16 = jnp.zeros((16,), jnp.float32)
        lane15 = jnp.full((16,), 15, jnp.int32)
        bufs = ((gdst0, gsrc0, rows0, gsem0), (gdst1, gsrc1, rows1, gsem1),
                (gdst2, gsrc2, rows2, gsem2), (gdst3, gsrc3, rows3, gsem3))
        NBUF = len(bufs)

        @plsc.parallel_loop(0, RPT * (D // 16), unroll=4)
        def zrow(i):
            rows0[i // 8, pl.ds((i % 8) * 16, 16)] = zeros16

        r0 = sid * RPT
        pltpu.sync_copy(rows0.at[pl.ds(0, RPT)], acc_l.at[pl.ds(r0, RPT)])
        pltpu.sync_copy(rows0.at[pl.ds(0, RPT)], acc_r.at[pl.ds(r0, RPT)])
        plsc.subcore_barrier()

        def process(x_hbm, e_hbm, acc_sh, deg_hbm):
            base = wid * EPW
            pltpu.sync_copy(e_hbm.at[pl.ds(E + base, EPW)], dstv)
            pltpu.sync_copy(e_hbm.at[pl.ds(base, EPW)], srcv)

            @plsc.parallel_loop(0, B // 16, unroll=4)
            def zdeg(i):
                degv[pl.ds(i * 16, 16)] = zeros16

            def compact(i, off):
                d = dstv[pl.ds(i * 16, 16)]
                s = srcv[pl.ds(i * 16, 16)]
                mask = d < B
                scan = plsc.cumsum(mask.astype(jnp.int32))
                pos = off + scan - 1
                plsc.store_scatter(mdst, [pos], d, mask=mask)
                plsc.store_scatter(msrc, [pos], s, mask=mask)
                plsc.addupdate_scatter(degv, [d], ones16, mask=mask)
                last = lax.gather(
                    scan, lane15[:, None],
                    lax.GatherDimensionNumbers(
                        offset_dims=(), collapsed_slice_dims=(0,),
                        start_index_map=(0,)),
                    slice_sizes=(1,),
                    mode=lax.GatherScatterMode.PROMISE_IN_BOUNDS)
                return off + last

            off = lax.fori_loop(0, NV, compact, jnp.zeros((16,), jnp.int32),
                                unroll=2)
            m = off[0]

            def pad(j, _):
                mdst[pl.ds(m + j * 16, 16)] = jnp.full((16,), JUNK, jnp.int32)
                msrc[pl.ds(m + j * 16, 16)] = jnp.zeros((16,), jnp.int32)
                return 0

            lax.fori_loop(0, G // 16, pad, 0)

            ng = (m + G - 1) // G

            def stage(g, gd, gs):
                @plsc.parallel_loop(0, G // 16, unroll=4)
                def cpy(j):
                    gd[pl.ds(j * 16, 16)] = mdst[pl.ds(g * G + j * 16, 16)]
                    gs[pl.ds(j * 16, 16)] = msrc[pl.ds(g * G + j * 16, 16)]

            for b in range(NBUF):
                gd, gs, rw, sem = bufs[b]

                @pl.when(b < ng)
                def _():
                    stage(b, gd, gs)
                    pltpu.async_copy(x_hbm.at[gs], rw, sem)

            def ring(p, _):
                for b in range(NBUF):
                    g = NBUF * p + b
                    gd, gs, rw, sem = bufs[b]

                    @pl.when(g < ng)
                    def _():
                        pltpu.make_async_copy(x_hbm.at[gs], rw, sem).wait()
                        pltpu.sync_copy(rw, acc_sh.at[gd], add=True)

                        @pl.when(g + NBUF < ng)
                        def _():
                            stage(g + NBUF, gd, gs)
                            pltpu.async_copy(x_hbm.at[gs], rw, sem)

                return 0

            lax.fori_loop(0, (ng + NBUF - 1) // NBUF, ring, 0)

            pltpu.sync_copy(degv, deg_hbm.at[pl.ds(wid * B, B)])

        process(xl_hbm, el_hbm, acc_l, degl_hbm)
        process(xr_hbm, er_hbm, acc_r, degr_hbm)
        plsc.subcore_barrier()

        o0 = sid * OPT
        ob = cid * B + o0
        pltpu.sync_copy(acc_l.at[pl.ds(o0, OPT)], accl_hbm.at[pl.ds(ob, OPT)])
        pltpu.sync_copy(acc_r.at[pl.ds(o0, OPT)], accr_hbm.at[pl.ds(ob, OPT)])

    return sc_kernel(x_l, e_l, x_r, e_r)


def _tc_body(xl, xr, accl, accr, degl, degr, ws, wn, lw, lb,
             logits_o, dist_o, embl_o, embr_o):
    ones_w = jnp.ones((NW, 1), jnp.float32)
    dims = (((0,), (0,)), ((), ()))
    dl = lax.dot_general(degl[...], ones_w, dims,
                         preferred_element_type=jnp.float32)
    dr = lax.dot_general(degr[...], ones_w, dims,
                         preferred_element_type=jnp.float32)
    aggl = accl[0:B, :] + accl[B:2 * B, :]
    aggr = accr[0:B, :] + accr[B:2 * B, :]
    meanl = aggl / jnp.maximum(dl, 1.0)
    meanr = aggr / jnp.maximum(dr, 1.0)
    embl = jax.nn.relu(
        jnp.dot(xl[...], ws[...], preferred_element_type=jnp.float32)
        + jnp.dot(meanl, wn[...], preferred_element_type=jnp.float32))
    embr = jax.nn.relu(
        jnp.dot(xr[...], ws[...], preferred_element_type=jnp.float32)
        + jnp.dot(meanr, wn[...], preferred_element_type=jnp.float32))
    dot = jnp.sum(embl * embr, axis=1, keepdims=True)
    nl = jnp.maximum(jnp.sqrt(jnp.sum(embl * embl, axis=1, keepdims=True)), 1e-8)
    nr = jnp.maximum(jnp.sqrt(jnp.sum(embr * embr, axis=1, keepdims=True)), 1e-8)
    dist = dot / (nl * nr)
    logits_o[...] = dist * lw[...] + lb[...]
    dist_o[...] = dist
    embl_o[...] = embl
    embr_o[...] = embr


def kernel(x_l, edge_index_l, x_r, edge_index_r, W_self, W_neigh, lin_W,
           lin_b, batch_size):
    del batch_size
    x_l = x_l.astype(jnp.float32)
    x_r = x_r.astype(jnp.float32)
    N = x_l.shape[0]
    E = edge_index_l.shape[1]
    el = edge_index_l.astype(jnp.int32).reshape(2 * E)
    er = edge_index_r.astype(jnp.int32).reshape(2 * E)

    accl, degl, accr, degr = _sc_aggregate(x_l, el, x_r, er)

    full = lambda s: pl.BlockSpec(s, lambda i: (0,) * len(s))
    logits, dist, embl, embr = pl.pallas_call(
        _tc_body,
        grid=(1,),
        out_shape=(
            jax.ShapeDtypeStruct((B, 2), jnp.float32),
            jax.ShapeDtypeStruct((B, 1), jnp.float32),
            jax.ShapeDtypeStruct((B, D), jnp.float32),
            jax.ShapeDtypeStruct((B, D), jnp.float32),
        ),
        out_specs=(full((B, 2)), full((B, 1)), full((B, D)), full((B, D))),
        in_specs=[
            full((B, D)), full((B, D)),
            full((NC * B, D)), full((NC * B, D)),
            full((NW, B)), full((NW, B)),
            full((D, D)), full((D, D)),
            full((1, 2)), full((1, 2)),
        ],
    )(x_l, x_r, accl, accr, degl.reshape(NW, B), degr.reshape(NW, B),
      W_self, W_neigh, lin_W, lin_b.reshape(1, 2))

    return (logits, dist.reshape(B), embl, embr)


# speedup vs baseline: 1.3004x; 1.0597x over previous
import functools

import numpy as np

import jax
import jax.numpy as jnp
from jax import lax
from jax.experimental import pallas as pl
from jax.experimental.pallas import tpu as pltpu
from jax.experimental.pallas import tpu_sc as plsc

B = 1024
D = 128
NC = 2
NS = 16
NW = NC * NS
G = 128
JUNK = B
ACC_ROWS = 1152
RPT = ACC_ROWS // NS
OPT = B // NS


def _sc_aggregate(x_l, e_l, x_r, e_r):
    E = e_l.shape[0] // 2
    EPW = E // NW
    NV = EPW // 16
    MAXM = EPW + G

    mesh = plsc.VectorSubcoreMesh(
        core_axis_name="c", subcore_axis_name="s",
        num_cores=NC, num_subcores=NS)

    @functools.partial(
        pl.kernel,
        out_type=(
            jax.ShapeDtypeStruct((NC * B, D), jnp.float32),
            jax.ShapeDtypeStruct((NW * B,), jnp.float32),
            jax.ShapeDtypeStruct((NC * B, D), jnp.float32),
            jax.ShapeDtypeStruct((NW * B,), jnp.float32),
        ),
        mesh=mesh,
        compiler_params=pltpu.CompilerParams(needs_layout_passes=False),
        scratch_types=[
            pltpu.VMEM((EPW,), jnp.int32),
            pltpu.VMEM((EPW,), jnp.int32),
            pltpu.VMEM((MAXM,), jnp.int32),
            pltpu.VMEM((MAXM,), jnp.int32),
            pltpu.VMEM((G,), jnp.int32),
            pltpu.VMEM((G,), jnp.int32),
            pltpu.VMEM((G, D), jnp.float32),
            pltpu.VMEM((G,), jnp.int32),
            pltpu.VMEM((G,), jnp.int32),
            pltpu.VMEM((G, D), jnp.float32),
            pltpu.VMEM((G,), jnp.int32),
            pltpu.VMEM((G,), jnp.int32),
            pltpu.VMEM((G, D), jnp.float32),
            pltpu.VMEM((G,), jnp.int32),
            pltpu.VMEM((G,), jnp.int32),
            pltpu.VMEM((G, D), jnp.float32),
            pltpu.VMEM((B,), jnp.float32),
            pltpu.VMEM((((EPW // 16 + 15) // 16) * 16,), jnp.int32),
            pltpu.VMEM((((EPW // 16 + 15) // 16) * 16,), jnp.int32),
            pltpu.VMEM_SHARED((ACC_ROWS, D), jnp.float32),
            pltpu.VMEM_SHARED((ACC_ROWS, D), jnp.float32),
            pltpu.SemaphoreType.DMA,
            pltpu.SemaphoreType.DMA,
            pltpu.SemaphoreType.DMA,
            pltpu.SemaphoreType.DMA,
        ],
    )
    def sc_kernel(xl_hbm, el_hbm, xr_hbm, er_hbm,
                  accl_hbm, degl_hbm, accr_hbm, degr_hbm,
                  dstv, srcv, mdst, msrc, gdst0, gsrc0, rows0,
                  gdst1, gsrc1, rows1, gdst2, gsrc2, rows2,
                  gdst3, gsrc3, rows3, degv, cnts, bases,
                  acc_l, acc_r, gsem0, gsem1, gsem2, gsem3):
        cid = lax.axis_index("c")
        sid = lax.axis_index("s")
        wid = sid * NC + cid

        ones16 = jnp.ones((16,), jnp.float32)
        zeros16 = jnp.zeros((16,), jnp.float32)
        lane15 = jnp.full((16,), 15, jnp.int32)
        bufs = ((gdst0, gsrc0, rows0, gsem0), (gdst1, gsrc1, rows1, gsem1),
                (gdst2, gsrc2, rows2, gsem2), (gdst3, gsrc3, rows3, gsem3))
        NBUF = len(bufs)

        @plsc.parallel_loop(0, RPT * (D // 16), unroll=4)
        def zrow(i):
            rows0[i // 8, pl.ds((i % 8) * 16, 16)] = zeros16

        r0 = sid * RPT
        pltpu.sync_copy(rows0.at[pl.ds(0, RPT)], acc_l.at[pl.ds(r0, RPT)])
        pltpu.sync_copy(rows0.at[pl.ds(0, RPT)], acc_r.at[pl.ds(r0, RPT)])
        plsc.subcore_barrier()

        def process(x_hbm, e_hbm, acc_sh, deg_hbm):
            base = wid * EPW
            pltpu.sync_copy(e_hbm.at[pl.ds(E + base, EPW)], dstv)
            pltpu.sync_copy(e_hbm.at[pl.ds(base, EPW)], srcv)

            @plsc.parallel_loop(0, B // 16, unroll=4)
            def zdeg(i):
                degv[pl.ds(i * 16, 16)] = zeros16

            NVC = (NV + 15) // 16
            iota16 = lax.iota(jnp.int32, 16)
            mask0 = iota16 == 0
            zi16 = jnp.zeros((16,), jnp.int32)
            cnts[pl.ds(16 * (NVC - 1), 16)] = zi16

            @plsc.parallel_loop(0, NV, unroll=4)
            def count(i):
                d = dstv[pl.ds(i * 16, 16)]
                c = plsc.all_reduce_population_count(d < B)
                plsc.store_scatter(cnts, [zi16 + i], c, mask=mask0)

            def scanchunk(j, carry):
                v = cnts[pl.ds(j * 16, 16)]
                sc = plsc.cumsum(v)
                bases[pl.ds(j * 16, 16)] = carry + sc - v
                last = lax.gather(
                    sc, lane15[:, None],
                    lax.GatherDimensionNumbers(
                        offset_dims=(), collapsed_slice_dims=(0,),
                        start_index_map=(0,)),
                    slice_sizes=(1,),
                    mode=lax.GatherScatterMode.PROMISE_IN_BOUNDS)
                return carry + last

            off = lax.fori_loop(0, NVC, scanchunk, jnp.zeros((16,), jnp.int32))
            m = off[0]

            @plsc.parallel_loop(0, NV, unroll=2)
            def emit(i):
                d = dstv[pl.ds(i * 16, 16)]
                s = srcv[pl.ds(i * 16, 16)]
                mask = d < B
                scan = plsc.cumsum(mask.astype(jnp.int32))
                base = plsc.load_gather(bases, [zi16 + i])
                pos = base + scan - 1
                plsc.store_scatter(mdst, [pos], d, mask=mask)
                plsc.store_scatter(msrc, [pos], s, mask=mask)
                plsc.addupdate_scatter(degv, [d], ones16, mask=mask)

            def pad(j, _):
                mdst[pl.ds(m + j * 16, 16)] = jnp.full((16,), JUNK, jnp.int32)
                msrc[pl.ds(m + j * 16, 16)] = jnp.zeros((16,), jnp.int32)
                return 0

            lax.fori_loop(0, G // 16, pad, 0)

            ng = (m + G - 1) // G

            def stage(g, gd, gs):
                @plsc.parallel_loop(0, G // 16, unroll=4)
                def cpy(j):
                    gd[pl.ds(j * 16, 16)] = mdst[pl.ds(g * G + j * 16, 16)]
                    gs[pl.ds(j * 16, 16)] = msrc[pl.ds(g * G + j * 16, 16)]

            for b in range(NBUF):
                gd, gs, rw, sem = bufs[b]

                @pl.when(b < ng)
                def _():
                    stage(b, gd, gs)
                    pltpu.async_copy(x_hbm.at[gs], rw, sem)

            def ring(p, _):
                for b in range(NBUF):
                    g = NBUF * p + b
                    gd, gs, rw, sem = bufs[b]

                    @pl.when(g < ng)
                    def _():
                        pltpu.make_async_copy(x_hbm.at[gs], rw, sem).wait()
                        pltpu.sync_copy(rw, acc_sh.at[gd], add=True)

                        @pl.when(g + NBUF < ng)
                        def _():
                            stage(g + NBUF, gd, gs)
                            pltpu.async_copy(x_hbm.at[gs], rw, sem)

                return 0

            lax.fori_loop(0, (ng + NBUF - 1) // NBUF, ring, 0)

            pltpu.sync_copy(degv, deg_hbm.at[pl.ds(wid * B, B)])

        process(xl_hbm, el_hbm, acc_l, degl_hbm)
        process(xr_hbm, er_hbm, acc_r, degr_hbm)
        plsc.subcore_barrier()

        o0 = sid * OPT
        ob = cid * B + o0
        pltpu.sync_copy(acc_l.at[pl.ds(o0, OPT)], accl_hbm.at[pl.ds(ob, OPT)])
        pltpu.sync_copy(acc_r.at[pl.ds(o0, OPT)], accr_hbm.at[pl.ds(ob, OPT)])

    return sc_kernel(x_l, e_l, x_r, e_r)


def _tc_body(xl, xr, accl, accr, degl, degr, ws, wn, lw, lb,
             logits_o, dist_o, embl_o, embr_o):
    ones_w = jnp.ones((NW, 1), jnp.float32)
    dims = (((0,), (0,)), ((), ()))
    dl = lax.dot_general(degl[...], ones_w, dims,
                         preferred_element_type=jnp.float32)
    dr = lax.dot_general(degr[...], ones_w, dims,
                         preferred_element_type=jnp.float32)
    aggl = accl[0:B, :] + accl[B:2 * B, :]
    aggr = accr[0:B, :] + accr[B:2 * B, :]
    meanl = aggl / jnp.maximum(dl, 1.0)
    meanr = aggr / jnp.maximum(dr, 1.0)
    embl = jax.nn.relu(
        jnp.dot(xl[...], ws[...], preferred_element_type=jnp.float32)
        + jnp.dot(meanl, wn[...], preferred_element_type=jnp.float32))
    embr = jax.nn.relu(
        jnp.dot(xr[...], ws[...], preferred_element_type=jnp.float32)
        + jnp.dot(meanr, wn[...], preferred_element_type=jnp.float32))
    dot = jnp.sum(embl * embr, axis=1, keepdims=True)
    nl = jnp.maximum(jnp.sqrt(jnp.sum(embl * embl, axis=1, keepdims=True)), 1e-8)
    nr = jnp.maximum(jnp.sqrt(jnp.sum(embr * embr, axis=1, keepdims=True)), 1e-8)
    dist = dot / (nl * nr)
    logits_o[...] = dist * lw[...] + lb[...]
    dist_o[...] = dist
    embl_o[...] = embl
    embr_o[...] = embr


def kernel(x_l, edge_index_l, x_r, edge_index_r, W_self, W_neigh, lin_W,
           lin_b, batch_size):
    del batch_size
    x_l = x_l.astype(jnp.float32)
    x_r = x_r.astype(jnp.float32)
    N = x_l.shape[0]
    E = edge_index_l.shape[1]
    el = edge_index_l.astype(jnp.int32).reshape(2 * E)
    er = edge_index_r.astype(jnp.int32).reshape(2 * E)

    accl, degl, accr, degr = _sc_aggregate(x_l, el, x_r, er)

    full = lambda s: pl.BlockSpec(s, lambda i: (0,) * len(s))
    logits, dist, embl, embr = pl.pallas_call(
        _tc_body,
        grid=(1,),
        out_shape=(
            jax.ShapeDtypeStruct((B, 2), jnp.float32),
            jax.ShapeDtypeStruct((B, 1), jnp.float32),
            jax.ShapeDtypeStruct((B, D), jnp.float32),
            jax.ShapeDtypeStruct((B, D), jnp.float32),
        ),
        out_specs=(full((B, 2)), full((B, 1)), full((B, D)), full((B, D))),
        in_specs=[
            full((B, D)), full((B, D)),
            full((NC * B, D)), full((NC * B, D)),
            full((NW, B)), full((NW, B)),
            full((D, D)), full((D, D)),
            full((1, 2)), full((1, 2)),
        ],
    )(x_l, x_r, accl, accr, degl.reshape(NW, B), degr.reshape(NW, B),
      W_self, W_neigh, lin_W, lin_b.reshape(1, 2))

    return (logits, dist.reshape(B), embl, embr)


# speedup vs baseline: 1.3922x; 1.0706x over previous
import functools

import numpy as np

import jax
import jax.numpy as jnp
from jax import lax
from jax.experimental import pallas as pl
from jax.experimental.pallas import tpu as pltpu
from jax.experimental.pallas import tpu_sc as plsc

B = 1024
D = 128
NC = 2
NS = 16
NW = NC * NS
G = 128
JUNK = B
ACC_ROWS = 1152
RPT = ACC_ROWS // NS
OPT = B // NS


def _sc_aggregate(x_l, e_l, x_r, e_r):
    E = e_l.shape[0] // 2
    EPW = E // NW
    NV = EPW // 16
    MAXM = EPW + G

    mesh = plsc.VectorSubcoreMesh(
        core_axis_name="c", subcore_axis_name="s",
        num_cores=NC, num_subcores=NS)

    @functools.partial(
        pl.kernel,
        out_type=(
            jax.ShapeDtypeStruct((NC * B, D), jnp.float32),
            jax.ShapeDtypeStruct((NW * B,), jnp.float32),
            jax.ShapeDtypeStruct((NC * B, D), jnp.float32),
            jax.ShapeDtypeStruct((NW * B,), jnp.float32),
        ),
        mesh=mesh,
        compiler_params=pltpu.CompilerParams(needs_layout_passes=False),
        scratch_types=[
            pltpu.VMEM((EPW,), jnp.int32),
            pltpu.VMEM((EPW,), jnp.int32),
            pltpu.VMEM((MAXM,), jnp.int32),
            pltpu.VMEM((MAXM,), jnp.int32),
            pltpu.VMEM((G,), jnp.int32),
            pltpu.VMEM((G,), jnp.int32),
            pltpu.VMEM((G, D), jnp.float32),
            pltpu.VMEM((G,), jnp.int32),
            pltpu.VMEM((G,), jnp.int32),
            pltpu.VMEM((G, D), jnp.float32),
            pltpu.VMEM((G,), jnp.int32),
            pltpu.VMEM((G,), jnp.int32),
            pltpu.VMEM((G, D), jnp.float32),
            pltpu.VMEM((G,), jnp.int32),
            pltpu.VMEM((G,), jnp.int32),
            pltpu.VMEM((G, D), jnp.float32),
            pltpu.VMEM((B,), jnp.float32),
            pltpu.VMEM((((EPW // 16 + 15) // 16) * 16,), jnp.int32),
            pltpu.VMEM((((EPW // 16 + 15) // 16) * 16,), jnp.int32),
            pltpu.VMEM_SHARED((ACC_ROWS, D), jnp.float32),
            pltpu.VMEM_SHARED((ACC_ROWS, D), jnp.float32),
            pltpu.SemaphoreType.DMA,
            pltpu.SemaphoreType.DMA,
            pltpu.SemaphoreType.DMA,
            pltpu.SemaphoreType.DMA,
            pltpu.SemaphoreType.DMA,
            pltpu.SemaphoreType.DMA,
        ],
    )
    def sc_kernel(xl_hbm, el_hbm, xr_hbm, er_hbm,
                  accl_hbm, degl_hbm, accr_hbm, degr_hbm,
                  dstv, srcv, mdst, msrc, gdst0, gsrc0, rows0,
                  gdst1, gsrc1, rows1, gdst2, gsrc2, rows2,
                  gdst3, gsrc3, rows3, degv, cnts, bases,
                  acc_l, acc_r, gsem0, gsem1, gsem2, gsem3, esem0, esem1):
        cid = lax.axis_index("c")
        sid = lax.axis_index("s")
        wid = sid * NC + cid
        base = wid * EPW

        def fire_edges(e_hbm):
            pltpu.async_copy(e_hbm.at[pl.ds(E + base, EPW)], dstv, esem0)
            pltpu.async_copy(e_hbm.at[pl.ds(base, EPW)], srcv, esem1)

        def wait_edges(e_hbm):
            pltpu.make_async_copy(
                e_hbm.at[pl.ds(E + base, EPW)], dstv, esem0).wait()
            pltpu.make_async_copy(
                e_hbm.at[pl.ds(base, EPW)], srcv, esem1).wait()

        fire_edges(el_hbm)

        ones16 = jnp.ones((16,), jnp.float32)
        zeros16 = jnp.zeros((16,), jnp.float32)
        lane15 = jnp.full((16,), 15, jnp.int32)
        bufs = ((gdst0, gsrc0, rows0, gsem0), (gdst1, gsrc1, rows1, gsem1),
                (gdst2, gsrc2, rows2, gsem2), (gdst3, gsrc3, rows3, gsem3))
        NBUF = len(bufs)

        @plsc.parallel_loop(0, RPT * (D // 16), unroll=4)
        def zrow(i):
            rows0[i // 8, pl.ds((i % 8) * 16, 16)] = zeros16

        r0 = sid * RPT
        pltpu.sync_copy(rows0.at[pl.ds(0, RPT)], acc_l.at[pl.ds(r0, RPT)])
        pltpu.sync_copy(rows0.at[pl.ds(0, RPT)], acc_r.at[pl.ds(r0, RPT)])
        plsc.subcore_barrier()

        def process(x_hbm, e_hbm, acc_sh, deg_hbm, e_next):
            wait_edges(e_hbm)

            @plsc.parallel_loop(0, B // 16, unroll=4)
            def zdeg(i):
                degv[pl.ds(i * 16, 16)] = zeros16

            NVC = (NV + 15) // 16
            iota16 = lax.iota(jnp.int32, 16)
            mask0 = iota16 == 0
            zi16 = jnp.zeros((16,), jnp.int32)
            cnts[pl.ds(16 * (NVC - 1), 16)] = zi16

            @plsc.parallel_loop(0, NV, unroll=4)
            def count(i):
                d = dstv[pl.ds(i * 16, 16)]
                c = plsc.all_reduce_population_count(d < B)
                plsc.store_scatter(cnts, [zi16 + i], c, mask=mask0)

            def scanchunk(j, carry):
                v = cnts[pl.ds(j * 16, 16)]
                sc = plsc.cumsum(v)
                bases[pl.ds(j * 16, 16)] = carry + sc - v
                last = lax.gather(
                    sc, lane15[:, None],
                    lax.GatherDimensionNumbers(
                        offset_dims=(), collapsed_slice_dims=(0,),
                        start_index_map=(0,)),
                    slice_sizes=(1,),
                    mode=lax.GatherScatterMode.PROMISE_IN_BOUNDS)
                return carry + last

            off = lax.fori_loop(0, NVC, scanchunk, jnp.zeros((16,), jnp.int32))
            m = off[0]

            @plsc.parallel_loop(0, NV, unroll=2)
            def emit(i):
                d = dstv[pl.ds(i * 16, 16)]
                s = srcv[pl.ds(i * 16, 16)]
                mask = d < B
                scan = plsc.cumsum(mask.astype(jnp.int32))
                base = plsc.load_gather(bases, [zi16 + i])
                pos = base + scan - 1
                plsc.store_scatter(mdst, [pos], d, mask=mask)
                plsc.store_scatter(msrc, [pos], s, mask=mask)
                plsc.addupdate_scatter(degv, [d], ones16, mask=mask)

            def pad(j, _):
                mdst[pl.ds(m + j * 16, 16)] = jnp.full((16,), JUNK, jnp.int32)
                msrc[pl.ds(m + j * 16, 16)] = jnp.zeros((16,), jnp.int32)
                return 0

            lax.fori_loop(0, G // 16, pad, 0)

            if e_next is not None:
                fire_edges(e_next)

            ng = (m + G - 1) // G

            def stage(g, gd, gs):
                @plsc.parallel_loop(0, G // 16, unroll=4)
                def cpy(j):
                    gd[pl.ds(j * 16, 16)] = mdst[pl.ds(g * G + j * 16, 16)]
                    gs[pl.ds(j * 16, 16)] = msrc[pl.ds(g * G + j * 16, 16)]

            for b in range(NBUF):
                gd, gs, rw, sem = bufs[b]

                @pl.when(b < ng)
                def _():
                    stage(b, gd, gs)
                    pltpu.async_copy(x_hbm.at[gs], rw, sem)

            def ring(p, _):
                for b in range(NBUF):
                    g = NBUF * p + b
                    gd, gs, rw, sem = bufs[b]

                    @pl.when(g < ng)
                    def _():
                        pltpu.make_async_copy(x_hbm.at[gs], rw, sem).wait()
                        pltpu.sync_copy(rw, acc_sh.at[gd], add=True)

                        @pl.when(g + NBUF < ng)
                        def _():
                            stage(g + NBUF, gd, gs)
                            pltpu.async_copy(x_hbm.at[gs], rw, sem)

                return 0

            lax.fori_loop(0, (ng + NBUF - 1) // NBUF, ring, 0)

            pltpu.sync_copy(degv, deg_hbm.at[pl.ds(wid * B, B)])

        process(xl_hbm, el_hbm, acc_l, degl_hbm, er_hbm)
        process(xr_hbm, er_hbm, acc_r, degr_hbm, None)
        plsc.subcore_barrier()

        o0 = sid * OPT
        ob = cid * B + o0
        pltpu.sync_copy(acc_l.at[pl.ds(o0, OPT)], accl_hbm.at[pl.ds(ob, OPT)])
        pltpu.sync_copy(acc_r.at[pl.ds(o0, OPT)], accr_hbm.at[pl.ds(ob, OPT)])

    return sc_kernel(x_l, e_l, x_r, e_r)


def _tc_body(xl, xr, accl, accr, degl, degr, ws, wn, lw, lb,
             logits_o, dist_o, embl_o, embr_o):
    ones_w = jnp.ones((NW, 1), jnp.float32)
    dims = (((0,), (0,)), ((), ()))
    dl = lax.dot_general(degl[...], ones_w, dims,
                         preferred_element_type=jnp.float32)
    dr = lax.dot_general(degr[...], ones_w, dims,
                         preferred_element_type=jnp.float32)
    aggl = accl[0:B, :] + accl[B:2 * B, :]
    aggr = accr[0:B, :] + accr[B:2 * B, :]
    meanl = aggl / jnp.maximum(dl, 1.0)
    meanr = aggr / jnp.maximum(dr, 1.0)
    embl = jax.nn.relu(
        jnp.dot(xl[...], ws[...], preferred_element_type=jnp.float32)
        + jnp.dot(meanl, wn[...], preferred_element_type=jnp.float32))
    embr = jax.nn.relu(
        jnp.dot(xr[...], ws[...], preferred_element_type=jnp.float32)
        + jnp.dot(meanr, wn[...], preferred_element_type=jnp.float32))
    dot = jnp.sum(embl * embr, axis=1, keepdims=True)
    nl = jnp.maximum(jnp.sqrt(jnp.sum(embl * embl, axis=1, keepdims=True)), 1e-8)
    nr = jnp.maximum(jnp.sqrt(jnp.sum(embr * embr, axis=1, keepdims=True)), 1e-8)
    dist = dot / (nl * nr)
    logits_o[...] = dist * lw[...] + lb[...]
    dist_o[...] = dist
    embl_o[...] = embl
    embr_o[...] = embr


def kernel(x_l, edge_index_l, x_r, edge_index_r, W_self, W_neigh, lin_W,
           lin_b, batch_size):
    del batch_size
    x_l = x_l.astype(jnp.float32)
    x_r = x_r.astype(jnp.float32)
    N = x_l.shape[0]
    E = edge_index_l.shape[1]
    el = edge_index_l.astype(jnp.int32).reshape(2 * E)
    er = edge_index_r.astype(jnp.int32).reshape(2 * E)

    accl, degl, accr, degr = _sc_aggregate(x_l, el, x_r, er)

    full = lambda s: pl.BlockSpec(s, lambda i: (0,) * len(s))
    logits, dist, embl, embr = pl.pallas_call(
        _tc_body,
        grid=(1,),
        out_shape=(
            jax.ShapeDtypeStruct((B, 2), jnp.float32),
            jax.ShapeDtypeStruct((B, 1), jnp.float32),
            jax.ShapeDtypeStruct((B, D), jnp.float32),
            jax.ShapeDtypeStruct((B, D), jnp.float32),
        ),
        out_specs=(full((B, 2)), full((B, 1)), full((B, D)), full((B, D))),
        in_specs=[
            full((B, D)), full((B, D)),
            full((NC * B, D)), full((NC * B, D)),
            full((NW, B)), full((NW, B)),
            full((D, D)), full((D, D)),
            full((1, 2)), full((1, 2)),
        ],
    )(x_l, x_r, accl, accr, degl.reshape(NW, B), degr.reshape(NW, B),
      W_self, W_neigh, lin_W, lin_b.reshape(1, 2))

    return (logits, dist.reshape(B), embl, embr)


# speedup vs baseline: 1.3983x; 1.0044x over previous
import functools

import jax
import jax.numpy as jnp
from jax import lax
from jax.experimental import pallas as pl
from jax.experimental.pallas import tpu as pltpu
from jax.experimental.pallas import tpu_sc as plsc

B = 1024
D = 128
NC = 2
NS = 16
NW = NC * NS
G = 128
JUNK = B
ACC_ROWS = 1152
RPT = ACC_ROWS // NS
OPT = B // NS


def _sc_aggregate(x_l, e_l, x_r, e_r):
    E = e_l.shape[0] // 2
    EPW = E // NW
    NV = EPW // 16
    MAXM = EPW + G

    mesh = plsc.VectorSubcoreMesh(
        core_axis_name="c", subcore_axis_name="s",
        num_cores=NC, num_subcores=NS)

    @functools.partial(
        pl.kernel,
        out_type=(
            jax.ShapeDtypeStruct((NC * B, D), jnp.float32),
            jax.ShapeDtypeStruct((NW * B,), jnp.float32),
            jax.ShapeDtypeStruct((NC * B, D), jnp.float32),
            jax.ShapeDtypeStruct((NW * B,), jnp.float32),
        ),
        mesh=mesh,
        compiler_params=pltpu.CompilerParams(needs_layout_passes=False),
        scratch_types=[
            pltpu.VMEM((EPW,), jnp.int32),
            pltpu.VMEM((EPW,), jnp.int32),
            pltpu.VMEM((MAXM,), jnp.int32),
            pltpu.VMEM((MAXM,), jnp.int32),
            pltpu.VMEM((G,), jnp.int32),
            pltpu.VMEM((G,), jnp.int32),
            pltpu.VMEM((G, D), jnp.float32),
            pltpu.VMEM((G,), jnp.int32),
            pltpu.VMEM((G,), jnp.int32),
            pltpu.VMEM((G, D), jnp.float32),
            pltpu.VMEM((G,), jnp.int32),
            pltpu.VMEM((G,), jnp.int32),
            pltpu.VMEM((G, D), jnp.float32),
            pltpu.VMEM((G,), jnp.int32),
            pltpu.VMEM((G,), jnp.int32),
            pltpu.VMEM((G, D), jnp.float32),
            pltpu.VMEM((B,), jnp.float32),
            pltpu.VMEM((((EPW // 16 + 15) // 16) * 16,), jnp.int32),
            pltpu.VMEM((((EPW // 16 + 15) // 16) * 16,), jnp.int32),
            pltpu.VMEM_SHARED((ACC_ROWS, D), jnp.float32),
            pltpu.VMEM_SHARED((ACC_ROWS, D), jnp.float32),
            pltpu.SemaphoreType.DMA,
            pltpu.SemaphoreType.DMA,
            pltpu.SemaphoreType.DMA,
            pltpu.SemaphoreType.DMA,
            pltpu.SemaphoreType.DMA,
            pltpu.SemaphoreType.DMA,
        ],
    )
    def sc_kernel(xl_hbm, el_hbm, xr_hbm, er_hbm,
                  accl_hbm, degl_hbm, accr_hbm, degr_hbm,
                  dstv, srcv, mdst, msrc, gdst0, gsrc0, rows0,
                  gdst1, gsrc1, rows1, gdst2, gsrc2, rows2,
                  gdst3, gsrc3, rows3, degv, cnts, bases,
                  acc_l, acc_r, gsem0, gsem1, gsem2, gsem3, esem0, esem1):
        cid = lax.axis_index("c")
        sid = lax.axis_index("s")
        wid = sid * NC + cid
        base = wid * EPW

        def fire_edges(e_hbm):
            pltpu.async_copy(e_hbm.at[pl.ds(E + base, EPW)], dstv, esem0)
            pltpu.async_copy(e_hbm.at[pl.ds(base, EPW)], srcv, esem1)

        def wait_edges(e_hbm):
            pltpu.make_async_copy(
                e_hbm.at[pl.ds(E + base, EPW)], dstv, esem0).wait()
            pltpu.make_async_copy(
                e_hbm.at[pl.ds(base, EPW)], srcv, esem1).wait()

        fire_edges(el_hbm)

        ones16 = jnp.ones((16,), jnp.float32)
        zeros16 = jnp.zeros((16,), jnp.float32)
        lane15 = jnp.full((16,), 15, jnp.int32)
        bufs = ((gdst0, gsrc0, rows0, gsem0), (gdst1, gsrc1, rows1, gsem1),
                (gdst2, gsrc2, rows2, gsem2), (gdst3, gsrc3, rows3, gsem3))
        NBUF = len(bufs)

        @plsc.parallel_loop(0, RPT * (D // 16), unroll=4)
        def zrow(i):
            rows0[i // 8, pl.ds((i % 8) * 16, 16)] = zeros16

        r0 = sid * RPT
        pltpu.sync_copy(rows0.at[pl.ds(0, RPT)], acc_l.at[pl.ds(r0, RPT)])
        pltpu.sync_copy(rows0.at[pl.ds(0, RPT)], acc_r.at[pl.ds(r0, RPT)])
        plsc.subcore_barrier()

        def process(x_hbm, e_hbm, acc_sh, deg_hbm, e_next):
            wait_edges(e_hbm)

            @plsc.parallel_loop(0, B // 16, unroll=4)
            def zdeg(i):
                degv[pl.ds(i * 16, 16)] = zeros16

            NVC = (NV + 15) // 16
            iota16 = lax.iota(jnp.int32, 16)
            mask0 = iota16 == 0
            zi16 = jnp.zeros((16,), jnp.int32)
            cnts[pl.ds(16 * (NVC - 1), 16)] = zi16

            @plsc.parallel_loop(0, NV, unroll=4)
            def count(i):
                d = dstv[pl.ds(i * 16, 16)]
                c = plsc.all_reduce_population_count(d < B)
                plsc.store_scatter(cnts, [zi16 + i], c, mask=mask0)

            def scanchunk(j, carry):
                v = cnts[pl.ds(j * 16, 16)]
                sc = plsc.cumsum(v)
                bases[pl.ds(j * 16, 16)] = carry + sc - v
                last = lax.gather(
                    sc, lane15[:, None],
                    lax.GatherDimensionNumbers(
                        offset_dims=(), collapsed_slice_dims=(0,),
                        start_index_map=(0,)),
                    slice_sizes=(1,),
                    mode=lax.GatherScatterMode.PROMISE_IN_BOUNDS)
                return carry + last

            off = lax.fori_loop(0, NVC, scanchunk, jnp.zeros((16,), jnp.int32))
            m = off[0]

            @plsc.parallel_loop(0, NV, unroll=2)
            def emit(i):
                d = dstv[pl.ds(i * 16, 16)]
                s = srcv[pl.ds(i * 16, 16)]
                mask = d < B
                scan = plsc.cumsum(mask.astype(jnp.int32))
                base = plsc.load_gather(bases, [zi16 + i])
                pos = base + scan - 1
                plsc.store_scatter(mdst, [pos], d, mask=mask)
                plsc.store_scatter(msrc, [pos], s, mask=mask)
                plsc.addupdate_scatter(degv, [d], ones16, mask=mask)

            def pad(j, _):
                mdst[pl.ds(m + j * 16, 16)] = jnp.full((16,), JUNK, jnp.int32)
                msrc[pl.ds(m + j * 16, 16)] = jnp.zeros((16,), jnp.int32)
                return 0

            lax.fori_loop(0, G // 16, pad, 0)

            if e_next is not None:
                fire_edges(e_next)

            ng = (m + G - 1) // G

            def stage(g, gd, gs):
                @plsc.parallel_loop(0, G // 16, unroll=4)
                def cpy(j):
                    gd[pl.ds(j * 16, 16)] = mdst[pl.ds(g * G + j * 16, 16)]
                    gs[pl.ds(j * 16, 16)] = msrc[pl.ds(g * G + j * 16, 16)]

            for b in range(NBUF):
                gd, gs, rw, sem = bufs[b]

                @pl.when(b < ng)
                def _():
                    stage(b, gd, gs)
                    pltpu.async_copy(x_hbm.at[gs], rw, sem)

            def ring(p, _):
                for b in range(NBUF):
                    g = NBUF * p + b
                    gd, gs, rw, sem = bufs[b]

                    @pl.when(g < ng)
                    def _():
                        pltpu.make_async_copy(x_hbm.at[gs], rw, sem).wait()
                        pltpu.sync_copy(rw, acc_sh.at[gd], add=True)

                        @pl.when(g + NBUF < ng)
                        def _():
                            stage(g + NBUF, gd, gs)
                            pltpu.async_copy(x_hbm.at[gs], rw, sem)

                return 0

            lax.fori_loop(0, (ng + NBUF - 1) // NBUF, ring, 0)

            pltpu.sync_copy(degv, deg_hbm.at[pl.ds(wid * B, B)])

        process(xl_hbm, el_hbm, acc_l, degl_hbm, er_hbm)
        process(xr_hbm, er_hbm, acc_r, degr_hbm, None)
        plsc.subcore_barrier()

        o0 = sid * OPT
        ob = cid * B + o0
        pltpu.sync_copy(acc_l.at[pl.ds(o0, OPT)], accl_hbm.at[pl.ds(ob, OPT)])
        pltpu.sync_copy(acc_r.at[pl.ds(o0, OPT)], accr_hbm.at[pl.ds(ob, OPT)])

    return sc_kernel(x_l, e_l, x_r, e_r)


def _tc_body(xl, xr, accl, accr, degl, degr, ws, wn, lw, lb,
             logits_o, dist_o, embl_o, embr_o):
    ones_w = jnp.ones((NW, 1), jnp.float32)
    dims = (((0,), (0,)), ((), ()))
    dl = lax.dot_general(degl[...], ones_w, dims,
                         preferred_element_type=jnp.float32)
    dr = lax.dot_general(degr[...], ones_w, dims,
                         preferred_element_type=jnp.float32)
    aggl = accl[0:B, :] + accl[B:2 * B, :]
    aggr = accr[0:B, :] + accr[B:2 * B, :]
    meanl = aggl / jnp.maximum(dl, 1.0)
    meanr = aggr / jnp.maximum(dr, 1.0)
    embl = jax.nn.relu(
        jnp.dot(xl[...], ws[...], preferred_element_type=jnp.float32)
        + jnp.dot(meanl, wn[...], preferred_element_type=jnp.float32))
    embr = jax.nn.relu(
        jnp.dot(xr[...], ws[...], preferred_element_type=jnp.float32)
        + jnp.dot(meanr, wn[...], preferred_element_type=jnp.float32))
    dot = jnp.sum(embl * embr, axis=1, keepdims=True)
    nl = jnp.maximum(jnp.sqrt(jnp.sum(embl * embl, axis=1, keepdims=True)), 1e-8)
    nr = jnp.maximum(jnp.sqrt(jnp.sum(embr * embr, axis=1, keepdims=True)), 1e-8)
    dist = dot / (nl * nr)
    logits_o[...] = dist * lw[...] + lb[...]
    dist_o[...] = dist
    embl_o[...] = embl
    embr_o[...] = embr


def kernel(x_l, edge_index_l, x_r, edge_index_r, W_self, W_neigh, lin_W,
           lin_b, batch_size):
    del batch_size
    x_l = x_l.astype(jnp.float32)
    x_r = x_r.astype(jnp.float32)
    E = edge_index_l.shape[1]
    el = edge_index_l.astype(jnp.int32).reshape(2 * E)
    er = edge_index_r.astype(jnp.int32).reshape(2 * E)

    accl, degl, accr, degr = _sc_aggregate(x_l, el, x_r, er)

    full = lambda s: pl.BlockSpec(s, lambda i: (0,) * len(s))
    logits, dist, embl, embr = pl.pallas_call(
        _tc_body,
        grid=(1,),
        out_shape=(
            jax.ShapeDtypeStruct((B, 2), jnp.float32),
            jax.ShapeDtypeStruct((B, 1), jnp.float32),
            jax.ShapeDtypeStruct((B, D), jnp.float32),
            jax.ShapeDtypeStruct((B, D), jnp.float32),
        ),
        out_specs=(full((B, 2)), full((B, 1)), full((B, D)), full((B, D))),
        in_specs=[
            full((B, D)), full((B, D)),
            full((NC * B, D)), full((NC * B, D)),
            full((NW, B)), full((NW, B)),
            full((D, D)), full((D, D)),
            full((1, 2)), full((1, 2)),
        ],
    )(x_l, x_r, accl, accr, degl.reshape(NW, B), degr.reshape(NW, B),
      W_self, W_neigh, lin_W, lin_b.reshape(1, 2))

    return (logits, dist.reshape(B), embl, embr)
